# baseline (speedup 1.0000x reference)
import functools

import jax
import jax.numpy as jnp
from jax import lax
from jax.experimental import pallas as pl
from jax.experimental.pallas import tpu as pltpu
from jax.experimental.pallas import tpu_sc as plsc

_VOCAB = 1000000
_EMB = 32
_B = 4096
_L = 200
_NW = 32
_RPW = _B // _NW
_NB = 4
_K = 2

_mesh = plsc.VectorSubcoreMesh(core_axis_name="c", subcore_axis_name="s")


@functools.partial(
    pl.kernel,
    mesh=_mesh,
    out_type=jax.ShapeDtypeStruct((_B, _L, _EMB), jnp.float32),
    scratch_types=[
        pltpu.VMEM((_RPW, _L), jnp.int32),
    ] + [pltpu.VMEM((_L, _EMB), jnp.float32) for _ in range(_NB)]
      + [pltpu.SemaphoreType.DMA for _ in range(2 * _NB)],
    compiler_params=pltpu.CompilerParams(use_tc_tiling_on_sc=False),
)
def _gather_kernel(idx_hbm, table_hbm, out_hbm, idx_v, *bufs_sems):
    bufs = bufs_sems[:_NB]
    gsem = bufs_sems[_NB:2 * _NB]
    ssem = bufs_sems[2 * _NB:]
    wid = lax.axis_index("s") * 2 + lax.axis_index("c")
    row0 = wid * _RPW
    pltpu.sync_copy(idx_hbm.at[pl.ds(row0, _RPW)], idx_v)

    def gather(r, slot):
        return pltpu.async_copy(
            table_hbm.at[idx_v.at[r]], bufs[slot], gsem[slot])

    def store(r, slot):
        return pltpu.async_copy(
            bufs[slot], out_hbm.at[row0 + r], ssem[slot])

    def wait_g(slot):
        pltpu.make_async_copy(table_hbm.at[idx_v.at[0]], bufs[slot],
                              gsem[slot]).wait()

    def wait_s(slot):
        pltpu.make_async_copy(bufs[slot], out_hbm.at[row0], ssem[slot]).wait()

    for b in range(_K):
        gather(b, b)

    for b in range(_NB):
        bg = (b + _K) % _NB
        if b + _K >= _NB:
            wait_s(bg)
        gather(b + _K, bg)
        wait_g(b)
        store(b, b)

    def body(rv, carry):
        for b in range(_NB):
            r = rv * _NB + b
            bg = (b + _K) % _NB
            wait_s(bg)
            gather(r + _K, bg)
            wait_g(b)
            store(r, b)
        return carry

    lax.fori_loop(1, _RPW // _NB - 1, body, 0)

    rv = _RPW // _NB - 1
    for b in range(_NB):
        r = rv * _NB + b
        bg = (b + _K) % _NB
        if r + _K < _RPW:
            wait_s(bg)
            gather(r + _K, bg)
        wait_g(b)
        store(r, b)

    for b in range(_NB):
        wait_s(b)


def kernel(input_ids, table):
    return _gather_kernel(input_ids, table)
